# baseline (speedup 1.0000x reference)
import functools

import jax
import jax.numpy as jnp
from jax import lax
from jax.experimental import pallas as pl
from jax.experimental.pallas import tpu as pltpu
from jax.experimental.pallas import tpu_sc as plsc

NC, NS = 2, 16
NW = NC * NS
B, QL, CL, D = 1024, 20, 200, 64
TOK = QL + CL
TOTAL = B * TOK
CHUNK = 128
ROWS_PER_W = TOTAL // NW
CHUNKS_PER_W = ROWS_PER_W // CHUNK

@functools.cache
def _build_gather_kernel():
    mesh = plsc.VectorSubcoreMesh(core_axis_name="c", subcore_axis_name="s")

    @functools.partial(
        pl.kernel,
        out_type=jax.ShapeDtypeStruct((TOTAL, D), jnp.float32),
        mesh=mesh,
        scratch_types=[
            pltpu.VMEM((CHUNKS_PER_W, CHUNK), jnp.int32),
            pltpu.VMEM((CHUNK, D), jnp.float32),
            pltpu.SemaphoreType.DMA,
        ],
        compiler_params=pltpu.CompilerParams(use_tc_tiling_on_sc=False),
    )
    def _gather_kernel(idx_hbm, table_hbm, out_hbm, idx_v, buf, sem):
        wid = lax.axis_index("s") * NC + lax.axis_index("c")
        row0 = wid * ROWS_PER_W
        pltpu.sync_copy(idx_hbm.at[wid], idx_v)

        def body(g, carry):
            pltpu.async_copy(table_hbm.at[idx_v.at[g]], buf, sem).wait()
            pltpu.sync_copy(buf, out_hbm.at[pl.ds(row0 + g * CHUNK, CHUNK)])
            return carry

        lax.fori_loop(0, CHUNKS_PER_W, body, 0)

    return _gather_kernel


def kernel(table, question_words, context_words):
    idx = jnp.concatenate(
        [question_words.astype(jnp.int32), context_words.astype(jnp.int32)],
        axis=1,
    ).reshape(NW, CHUNKS_PER_W, CHUNK)
    out = _build_gather_kernel()(idx, table)
    return out.reshape(B, TOK, D)


# speedup vs baseline: 1.0496x; 1.0496x over previous
import functools

import jax
import jax.numpy as jnp
from jax import lax
from jax.experimental import pallas as pl
from jax.experimental.pallas import tpu as pltpu
from jax.experimental.pallas import tpu_sc as plsc

NC, NS = 2, 16
NW = NC * NS
B, QL, CL, D = 1024, 20, 200, 64
TOK = QL + CL
TOTAL = B * TOK
CHUNK = 128
ROWS_PER_W = TOTAL // NW
CHUNKS_PER_W = ROWS_PER_W // CHUNK
NBUF = 5
OUTER = CHUNKS_PER_W // NBUF

@functools.cache
def _build_gather_kernel():
    mesh = plsc.VectorSubcoreMesh(core_axis_name="c", subcore_axis_name="s")

    @functools.partial(
        pl.kernel,
        out_type=jax.ShapeDtypeStruct((TOTAL, D), jnp.float32),
        mesh=mesh,
        scratch_types=[
            pltpu.VMEM((CHUNKS_PER_W, CHUNK), jnp.int32),
            pltpu.VMEM((NBUF, CHUNK, D), jnp.float32),
            pltpu.SemaphoreType.DMA((NBUF,)),
            pltpu.SemaphoreType.DMA((NBUF,)),
        ],
        compiler_params=pltpu.CompilerParams(use_tc_tiling_on_sc=False),
    )
    def _gather_kernel(idx_hbm, table_hbm, out_hbm, idx_v, bufs, gsem, wsem):
        wid = lax.axis_index("s") * NC + lax.axis_index("c")
        row0 = wid * ROWS_PER_W
        pltpu.sync_copy(idx_hbm.at[wid], idx_v)

        def outer(o, carry):
            base = o * NBUF
            gathers = []
            for b in range(NBUF):
                @pl.when(o > 0)
                def _(b=b):
                    pltpu.make_async_copy(
                        bufs.at[b], out_hbm.at[pl.ds(0, CHUNK)], wsem.at[b]
                    ).wait()

                gathers.append(
                    pltpu.async_copy(
                        table_hbm.at[idx_v.at[base + b]], bufs.at[b],
                        gsem.at[b],
                    )
                )
            for b in range(NBUF):
                gathers[b].wait()
                pltpu.async_copy(
                    bufs.at[b],
                    out_hbm.at[pl.ds(row0 + (base + b) * CHUNK, CHUNK)],
                    wsem.at[b],
                )
            return carry

        lax.fori_loop(0, OUTER, outer, 0)
        for b in range(NBUF):
            pltpu.make_async_copy(
                bufs.at[b], out_hbm.at[pl.ds(0, CHUNK)], wsem.at[b]
            ).wait()

    return _gather_kernel


def kernel(table, question_words, context_words):
    idx = jnp.concatenate(
        [question_words.astype(jnp.int32), context_words.astype(jnp.int32)],
        axis=1,
    ).reshape(NW, CHUNKS_PER_W, CHUNK)
    out = _build_gather_kernel()(idx, table)
    return out.reshape(B, TOK, D)


# speedup vs baseline: 1.0516x; 1.0019x over previous
import functools

import jax
import jax.numpy as jnp
from jax import lax
from jax.experimental import pallas as pl
from jax.experimental.pallas import tpu as pltpu
from jax.experimental.pallas import tpu_sc as plsc

NC, NS = 2, 16
NW = NC * NS
B, QL, CL, D = 1024, 20, 200, 64
TOK = QL + CL
TOTAL = B * TOK
CHUNK = 128
ROWS_PER_W = TOTAL // NW
CHUNKS_PER_W = ROWS_PER_W // CHUNK
NBUF = 11
OUTER = CHUNKS_PER_W // NBUF

@functools.cache
def _build_gather_kernel():
    mesh = plsc.VectorSubcoreMesh(core_axis_name="c", subcore_axis_name="s")

    @functools.partial(
        pl.kernel,
        out_type=jax.ShapeDtypeStruct((TOTAL, D), jnp.float32),
        mesh=mesh,
        scratch_types=[
            pltpu.VMEM((CHUNKS_PER_W, CHUNK), jnp.int32),
            pltpu.VMEM((NBUF, CHUNK, D), jnp.float32),
            pltpu.SemaphoreType.DMA((NBUF,)),
            pltpu.SemaphoreType.DMA((NBUF,)),
        ],
        compiler_params=pltpu.CompilerParams(use_tc_tiling_on_sc=False),
    )
    def _gather_kernel(idx_hbm, table_hbm, out_hbm, idx_v, bufs, gsem, wsem):
        wid = lax.axis_index("s") * NC + lax.axis_index("c")
        row0 = wid * ROWS_PER_W
        pltpu.sync_copy(idx_hbm.at[wid], idx_v)

        def outer(o, carry):
            base = o * NBUF
            gathers = []
            for b in range(NBUF):
                @pl.when(o > 0)
                def _(b=b):
                    pltpu.make_async_copy(
                        bufs.at[b], out_hbm.at[pl.ds(0, CHUNK)], wsem.at[b]
                    ).wait()

                gathers.append(
                    pltpu.async_copy(
                        table_hbm.at[idx_v.at[base + b]], bufs.at[b],
                        gsem.at[b],
                    )
                )
            for b in range(NBUF):
                gathers[b].wait()
                pltpu.async_copy(
                    bufs.at[b],
                    out_hbm.at[pl.ds(row0 + (base + b) * CHUNK, CHUNK)],
                    wsem.at[b],
                )
            return carry

        lax.fori_loop(0, OUTER, outer, 0)
        for b in range(NBUF):
            pltpu.make_async_copy(
                bufs.at[b], out_hbm.at[pl.ds(0, CHUNK)], wsem.at[b]
            ).wait()

    return _gather_kernel


def kernel(table, question_words, context_words):
    idx = jnp.concatenate(
        [question_words.astype(jnp.int32), context_words.astype(jnp.int32)],
        axis=1,
    ).reshape(NW, CHUNKS_PER_W, CHUNK)
    out = _build_gather_kernel()(idx, table)
    return out.reshape(B, TOK, D)
